# baseline (speedup 1.0000x reference)
import functools

import jax
import jax.numpy as jnp
from jax.experimental import pallas as pl
from jax.experimental.pallas import tpu as pltpu

BI = 256
BK = 2048


def _diffusion_kernel(theta_ref, t_ref, x_ref, a_ref, alpha_ref, w_ref,
                      b_ref, out_ref, acc_ref, *, nk):
    k = pl.program_id(1)

    @pl.when(k == 0)
    def _():
        acc_ref[...] = jnp.zeros_like(acc_ref)

    th = theta_ref[0]
    t = t_ref[...]
    q = th[0] * t[0]
    for s in range(1, t.shape[0]):
        q += th[s] * t[s]
    q = q * a_ref[...]
    xk = x_ref[pl.ds(k * BK, BK), :]
    acc_ref[...] += jnp.dot(q, xk, preferred_element_type=jnp.float32)

    @pl.when(k == nk - 1)
    def _():
        h = acc_ref[...]
        h = jnp.where(h >= 0, h, alpha_ref[...] * h)
        out = jax.lax.dot_general(
            h, w_ref[...], (((1,), (1,)), ((), ())),
            preferred_element_type=jnp.float32)
        out_ref[...] = out + b_ref[...]


@jax.jit
def kernel(theta, T_slices, x, a, alpha, W, b):
    S, N, _ = T_slices.shape
    D_in = x.shape[1]
    D_out = W.shape[0]
    ni, nk = N // BI, N // BK

    grid = (ni, nk)
    out = pl.pallas_call(
        functools.partial(_diffusion_kernel, nk=nk),
        grid=grid,
        in_specs=[
            pl.BlockSpec((1, S), lambda i, k: (0, 0)),
            pl.BlockSpec((S, BI, BK), lambda i, k: (0, i, k)),
            pl.BlockSpec((N, D_in), lambda i, k: (0, 0)),
            pl.BlockSpec((BI, BK), lambda i, k: (i, k)),
            pl.BlockSpec((1, D_in), lambda i, k: (0, 0)),
            pl.BlockSpec((D_out, D_in), lambda i, k: (0, 0)),
            pl.BlockSpec((1, D_out), lambda i, k: (0, 0)),
        ],
        out_specs=pl.BlockSpec((BI, D_out), lambda i, k: (i, 0)),
        out_shape=jax.ShapeDtypeStruct((N, D_out), jnp.float32),
        scratch_shapes=[pltpu.VMEM((BI, D_in), jnp.float32)],
        compiler_params=pltpu.CompilerParams(
            dimension_semantics=("parallel", "arbitrary")),
    )(theta.reshape(1, S), T_slices, x, a, alpha.reshape(1, D_in), W,
      b.reshape(1, D_out))
    return out


# speedup vs baseline: 1.0583x; 1.0583x over previous
import functools

import jax
import jax.numpy as jnp
from jax.experimental import pallas as pl
from jax.experimental.pallas import tpu as pltpu

BI = 1024
BK = 512


def _diffusion_kernel(theta_ref, t_ref, x_ref, a_ref, alpha_ref, w_ref,
                      b_ref, out_ref, acc_ref, *, nk):
    k = pl.program_id(1)

    @pl.when(k == 0)
    def _():
        acc_ref[...] = jnp.zeros_like(acc_ref)

    th = theta_ref[0]
    t = t_ref[...]
    q = th[0] * t[0]
    for s in range(1, t.shape[0]):
        q += th[s] * t[s]
    q = q * a_ref[...]
    xk = x_ref[pl.ds(k * BK, BK), :]
    acc_ref[...] += jnp.dot(q, xk, preferred_element_type=jnp.float32)

    @pl.when(k == nk - 1)
    def _():
        h = acc_ref[...]
        h = jnp.where(h >= 0, h, alpha_ref[...] * h)
        out = jax.lax.dot_general(
            h, w_ref[...], (((1,), (1,)), ((), ())),
            preferred_element_type=jnp.float32)
        out_ref[...] = out + b_ref[...]


@jax.jit
def kernel(theta, T_slices, x, a, alpha, W, b):
    S, N, _ = T_slices.shape
    D_in = x.shape[1]
    D_out = W.shape[0]
    ni, nk = N // BI, N // BK

    grid = (ni, nk)
    out = pl.pallas_call(
        functools.partial(_diffusion_kernel, nk=nk),
        grid=grid,
        in_specs=[
            pl.BlockSpec((1, S), lambda i, k: (0, 0)),
            pl.BlockSpec((S, BI, BK), lambda i, k: (0, i, k)),
            pl.BlockSpec((N, D_in), lambda i, k: (0, 0)),
            pl.BlockSpec((BI, BK), lambda i, k: (i, k)),
            pl.BlockSpec((1, D_in), lambda i, k: (0, 0)),
            pl.BlockSpec((D_out, D_in), lambda i, k: (0, 0)),
            pl.BlockSpec((1, D_out), lambda i, k: (0, 0)),
        ],
        out_specs=pl.BlockSpec((BI, D_out), lambda i, k: (i, 0)),
        out_shape=jax.ShapeDtypeStruct((N, D_out), jnp.float32),
        scratch_shapes=[pltpu.VMEM((BI, D_in), jnp.float32)],
        compiler_params=pltpu.CompilerParams(
            dimension_semantics=("parallel", "arbitrary")),
    )(theta.reshape(1, S), T_slices, x, a, alpha.reshape(1, D_in), W,
      b.reshape(1, D_out))
    return out


# speedup vs baseline: 1.0644x; 1.0058x over previous
import functools

import jax
import jax.numpy as jnp
from jax.experimental import pallas as pl
from jax.experimental.pallas import tpu as pltpu

BI = 512
BK = 1024


def _diffusion_kernel(theta_ref, t_ref, x_ref, a_ref, alpha_ref, w_ref,
                      b_ref, out_ref, acc_ref, *, nk):
    k = pl.program_id(1)

    @pl.when(k == 0)
    def _():
        acc_ref[...] = jnp.zeros_like(acc_ref)

    th = theta_ref[0]
    t = t_ref[...]
    q = th[0] * t[0]
    for s in range(1, t.shape[0]):
        q += th[s] * t[s]
    q = q * a_ref[...]
    xk = x_ref[pl.ds(k * BK, BK), :]
    acc_ref[...] += jnp.dot(q, xk, preferred_element_type=jnp.float32)

    @pl.when(k == nk - 1)
    def _():
        h = acc_ref[...]
        h = jnp.where(h >= 0, h, alpha_ref[...] * h)
        out = jax.lax.dot_general(
            h, w_ref[...], (((1,), (1,)), ((), ())),
            preferred_element_type=jnp.float32)
        out_ref[...] = out + b_ref[...]


@jax.jit
def kernel(theta, T_slices, x, a, alpha, W, b):
    S, N, _ = T_slices.shape
    D_in = x.shape[1]
    D_out = W.shape[0]
    ni, nk = N // BI, N // BK

    grid = (ni, nk)
    out = pl.pallas_call(
        functools.partial(_diffusion_kernel, nk=nk),
        grid=grid,
        in_specs=[
            pl.BlockSpec((1, S), lambda i, k: (0, 0)),
            pl.BlockSpec((S, BI, BK), lambda i, k: (0, i, k)),
            pl.BlockSpec((N, D_in), lambda i, k: (0, 0)),
            pl.BlockSpec((BI, BK), lambda i, k: (i, k)),
            pl.BlockSpec((1, D_in), lambda i, k: (0, 0)),
            pl.BlockSpec((D_out, D_in), lambda i, k: (0, 0)),
            pl.BlockSpec((1, D_out), lambda i, k: (0, 0)),
        ],
        out_specs=pl.BlockSpec((BI, D_out), lambda i, k: (i, 0)),
        out_shape=jax.ShapeDtypeStruct((N, D_out), jnp.float32),
        scratch_shapes=[pltpu.VMEM((BI, D_in), jnp.float32)],
        compiler_params=pltpu.CompilerParams(
            dimension_semantics=("parallel", "arbitrary")),
    )(theta.reshape(1, S), T_slices, x, a, alpha.reshape(1, D_in), W,
      b.reshape(1, D_out))
    return out
